# Initial kernel scaffold; baseline (speedup 1.0000x reference)
import jax
import jax.numpy as jnp
from jax.experimental import pallas as pl


def kernel(t, beta_t, alpha_t):
    raise NotImplementedError("write your pallas kernel here")


# speedup vs baseline: 8.3090x; 8.3090x over previous
import functools

import jax
import jax.numpy as jnp
from jax import lax
from jax.experimental import pallas as pl
from jax.experimental.pallas import tpu as pltpu
from jax.experimental.pallas import tpu_sc as plsc

NUM_STEPS_PAD = 1024
BATCH = 16384
NC = 2
NS = 16
NW = NC * NS
LANES = 16
B_PER_W = BATCH // NW


@functools.partial(
    pl.kernel,
    mesh=plsc.VectorSubcoreMesh(core_axis_name="c", subcore_axis_name="s"),
    compiler_params=pltpu.CompilerParams(needs_layout_passes=False),
    out_type=(
        jax.ShapeDtypeStruct((BATCH,), jnp.float32),
        jax.ShapeDtypeStruct((BATCH,), jnp.float32),
    ),
    scratch_types=[
        pltpu.VMEM((B_PER_W,), jnp.int32),
        pltpu.VMEM((NUM_STEPS_PAD,), jnp.float32),
        pltpu.VMEM((NUM_STEPS_PAD,), jnp.float32),
        pltpu.VMEM((B_PER_W,), jnp.float32),
        pltpu.VMEM((B_PER_W,), jnp.float32),
    ],
)
def _gather_sc(t_hbm, beta_hbm, alpha_hbm, alpha_out, beta_out,
               idx_v, beta_v, alpha_v, oa_v, ob_v):
    wid = lax.axis_index("s") * NC + lax.axis_index("c")
    base = wid * B_PER_W
    pltpu.sync_copy(t_hbm.at[pl.ds(base, B_PER_W)], idx_v)
    pltpu.sync_copy(beta_hbm, beta_v)
    pltpu.sync_copy(alpha_hbm, alpha_v)
    for i in range(B_PER_W // LANES):
        sl = pl.ds(i * LANES, LANES)
        idx = idx_v[sl]
        oa_v[sl] = plsc.load_gather(alpha_v, [idx])
        ob_v[sl] = plsc.load_gather(beta_v, [idx])
    pltpu.sync_copy(oa_v, alpha_out.at[pl.ds(base, B_PER_W)])
    pltpu.sync_copy(ob_v, beta_out.at[pl.ds(base, B_PER_W)])


def kernel(t, beta_t, alpha_t):
    pad = NUM_STEPS_PAD - beta_t.shape[0]
    beta_p = jnp.pad(beta_t.astype(jnp.float32), (0, pad))
    alpha_p = jnp.pad(alpha_t.astype(jnp.float32), (0, pad))
    alpha_g, beta_g = _gather_sc(t.astype(jnp.int32), beta_p, alpha_p)
    return alpha_g, beta_g


# speedup vs baseline: 8.3890x; 1.0096x over previous
import functools

import jax
import jax.numpy as jnp
from jax import lax
from jax.experimental import pallas as pl
from jax.experimental.pallas import tpu as pltpu
from jax.experimental.pallas import tpu_sc as plsc

NUM_STEPS = 1000
BATCH = 16384
NC = 2
NS = 16
NW = NC * NS
LANES = 16
B_PER_W = BATCH // NW


@functools.partial(
    pl.kernel,
    mesh=plsc.VectorSubcoreMesh(core_axis_name="c", subcore_axis_name="s"),
    compiler_params=pltpu.CompilerParams(needs_layout_passes=False),
    out_type=(
        jax.ShapeDtypeStruct((BATCH,), jnp.float32),
        jax.ShapeDtypeStruct((BATCH,), jnp.float32),
    ),
    scratch_types=[
        pltpu.VMEM((B_PER_W,), jnp.int32),
        pltpu.VMEM((NUM_STEPS,), jnp.float32),
        pltpu.VMEM((NUM_STEPS,), jnp.float32),
        pltpu.VMEM((B_PER_W,), jnp.float32),
        pltpu.VMEM((B_PER_W,), jnp.float32),
        pltpu.SemaphoreType.DMA,
        pltpu.SemaphoreType.DMA,
        pltpu.SemaphoreType.DMA,
    ],
)
def _gather_sc(t_hbm, beta_hbm, alpha_hbm, alpha_out, beta_out,
               idx_v, beta_v, alpha_v, oa_v, ob_v, sem0, sem1, sem2):
    wid = lax.axis_index("s") * NC + lax.axis_index("c")
    base = wid * B_PER_W
    cp_idx = pltpu.make_async_copy(t_hbm.at[pl.ds(base, B_PER_W)], idx_v, sem0)
    cp_beta = pltpu.make_async_copy(beta_hbm, beta_v, sem1)
    cp_alpha = pltpu.make_async_copy(alpha_hbm, alpha_v, sem2)
    cp_idx.start()
    cp_beta.start()
    cp_alpha.start()
    cp_idx.wait()
    cp_beta.wait()
    cp_alpha.wait()
    for i in range(B_PER_W // LANES):
        sl = pl.ds(i * LANES, LANES)
        idx = idx_v[sl]
        oa_v[sl] = plsc.load_gather(alpha_v, [idx])
        ob_v[sl] = plsc.load_gather(beta_v, [idx])
    cp_oa = pltpu.make_async_copy(oa_v, alpha_out.at[pl.ds(base, B_PER_W)], sem0)
    cp_ob = pltpu.make_async_copy(ob_v, beta_out.at[pl.ds(base, B_PER_W)], sem1)
    cp_oa.start()
    cp_ob.start()
    cp_oa.wait()
    cp_ob.wait()


def kernel(t, beta_t, alpha_t):
    alpha_g, beta_g = _gather_sc(t.astype(jnp.int32),
                                 beta_t.astype(jnp.float32),
                                 alpha_t.astype(jnp.float32))
    return alpha_g, beta_g


# speedup vs baseline: 8.3978x; 1.0010x over previous
import functools

import jax
import jax.numpy as jnp
from jax import lax
from jax.experimental import pallas as pl
from jax.experimental.pallas import tpu as pltpu
from jax.experimental.pallas import tpu_sc as plsc

NUM_STEPS = 1000
BATCH = 16384
NC = 2
NS = 16
NW = NC * NS
LANES = 16
B_PER_W = BATCH // NW


@functools.partial(
    pl.kernel,
    mesh=plsc.VectorSubcoreMesh(core_axis_name="c", subcore_axis_name="s"),
    compiler_params=pltpu.CompilerParams(needs_layout_passes=False),
    out_type=(
        jax.ShapeDtypeStruct((BATCH,), jnp.float32),
        jax.ShapeDtypeStruct((BATCH,), jnp.float32),
    ),
    scratch_types=[
        pltpu.VMEM((B_PER_W,), jnp.int32),
        pltpu.VMEM((NUM_STEPS,), jnp.float32),
        pltpu.VMEM((NUM_STEPS,), jnp.float32),
        pltpu.VMEM((B_PER_W,), jnp.float32),
        pltpu.VMEM((B_PER_W,), jnp.float32),
        pltpu.SemaphoreType.DMA,
        pltpu.SemaphoreType.DMA,
        pltpu.SemaphoreType.DMA,
    ],
)
def _gather_sc(t_hbm, beta_hbm, alpha_hbm, alpha_out, beta_out,
               idx_v, beta_v, alpha_v, oa_v, ob_v, sem0, sem1, sem2):
    wid = lax.axis_index("s") * NC + lax.axis_index("c")
    base = wid * B_PER_W
    cp_idx = pltpu.make_async_copy(t_hbm.at[pl.ds(base, B_PER_W)], idx_v, sem0)
    cp_beta = pltpu.make_async_copy(beta_hbm, beta_v, sem1)
    cp_alpha = pltpu.make_async_copy(alpha_hbm, alpha_v, sem2)
    cp_idx.start()
    cp_beta.start()
    cp_alpha.start()
    cp_idx.wait()
    cp_beta.wait()
    cp_alpha.wait()
    def body(i, carry):
        sl = pl.ds(i * LANES, LANES)
        idx = idx_v[sl]
        oa_v[sl] = plsc.load_gather(alpha_v, [idx])
        ob_v[sl] = plsc.load_gather(beta_v, [idx])
        return carry

    lax.fori_loop(0, B_PER_W // LANES, body, 0, unroll=4)
    cp_oa = pltpu.make_async_copy(oa_v, alpha_out.at[pl.ds(base, B_PER_W)], sem0)
    cp_ob = pltpu.make_async_copy(ob_v, beta_out.at[pl.ds(base, B_PER_W)], sem1)
    cp_oa.start()
    cp_ob.start()
    cp_oa.wait()
    cp_ob.wait()


def kernel(t, beta_t, alpha_t):
    alpha_g, beta_g = _gather_sc(t.astype(jnp.int32),
                                 beta_t.astype(jnp.float32),
                                 alpha_t.astype(jnp.float32))
    return alpha_g, beta_g
